# baseline (speedup 1.0000x reference)
import functools

import jax
import jax.numpy as jnp
from jax import lax
from jax.experimental import pallas as pl
from jax.experimental.pallas import tpu as pltpu
from jax.experimental.pallas import tpu_sc as plsc

N = 10000
E = 320000
D = 128

NC = 2
NS = 16
NW = NC * NS

G = 128
CHB = 8
RW = 80
RP = RW * NW
EP = RP * G

N_PAD = 10240
ZR = 16
ROWS_PER_SUB = N_PAD // NS


def _sc_agg_body(x_hbm, srcm_hbm, dstm_hbm, out_hbm,
                 src_v, dst_v, msgs_v, zbuf_v, acc, sem, isem):
    c = lax.axis_index("c")
    s = lax.axis_index("s")
    wid = s * NC + c

    zero = jnp.zeros((16,), jnp.float32)
    for i in range(ZR):
        for j in range(D // 16):
            zbuf_v[i, pl.ds(j * 16, 16)] = zero

    def zbody(i, carry):
        pltpu.sync_copy(zbuf_v, acc.at[pl.ds(s * ROWS_PER_SUB + i * ZR, ZR)])
        return carry
    lax.fori_loop(0, ROWS_PER_SUB // ZR, zbody, 0)

    plsc.subcore_barrier()

    def ebody(j, carry):
        row = wid * RW + j
        pltpu.sync_copy(srcm_hbm.at[row], src_v)
        pltpu.sync_copy(dstm_hbm.at[row], dst_v)
        pltpu.sync_copy(x_hbm.at[src_v], msgs_v)
        pltpu.sync_copy(msgs_v, acc.at[dst_v], add=True)
        return carry
    lax.fori_loop(0, RW, ebody, 0)

    plsc.subcore_barrier()

    base = c * N_PAD + s * ROWS_PER_SUB
    pltpu.sync_copy(acc.at[pl.ds(s * ROWS_PER_SUB, ROWS_PER_SUB)],
                    out_hbm.at[pl.ds(base, ROWS_PER_SUB)])


def _sc_agg(x, srcm, dstm):
    mesh = plsc.VectorSubcoreMesh(core_axis_name="c", subcore_axis_name="s",
                                  num_cores=NC, num_subcores=NS)
    return pl.kernel(
        _sc_agg_body,
        out_type=jax.ShapeDtypeStruct((NC * N_PAD, D), jnp.float32),
        mesh=mesh,
        scratch_types=[
            pltpu.VMEM((G,), jnp.int32),
            pltpu.VMEM((G,), jnp.int32),
            pltpu.VMEM((G, D), jnp.float32),
            pltpu.VMEM((ZR, D), jnp.float32),
            pltpu.VMEM_SHARED((N_PAD, D), jnp.float32),
            pltpu.SemaphoreType.DMA,
            pltpu.SemaphoreType.DMA,
        ],
    )(x, srcm, dstm)


def _mlp_body(x_ref, p0_ref, p1_ref, w1_ref, b1_ref, w2_ref, b2_ref, o_ref,
              *, relu_out):
    h = x_ref[...] + p0_ref[...] + p1_ref[...]
    h = jnp.dot(h, w1_ref[...], preferred_element_type=jnp.float32) + b1_ref[...]
    h = jnp.maximum(h, 0.0)
    h = jnp.dot(h, w2_ref[...], preferred_element_type=jnp.float32) + b2_ref[...]
    if relu_out:
        h = jnp.maximum(h, 0.0)
    o_ref[...] = h


def _mlp(x, p0, p1, W_a, b_a, W_b, b_b, relu_out):
    BR = 2000
    row_spec = pl.BlockSpec((BR, D), lambda i: (i, 0))
    full = pl.BlockSpec((D, D), lambda i: (0, 0))
    vec = pl.BlockSpec((1, D), lambda i: (0, 0))
    return pl.pallas_call(
        functools.partial(_mlp_body, relu_out=relu_out),
        grid=(N // BR,),
        in_specs=[row_spec, row_spec, row_spec, full, vec, full, vec],
        out_specs=row_spec,
        out_shape=jax.ShapeDtypeStruct((N, D), jnp.float32),
    )(x, p0, p1, W_a, b_a.reshape(1, D), W_b, b_b.reshape(1, D))


def kernel(x, edge_index, W1, b1, W2, b2, W3, b3, W4, b4):
    src = edge_index[0]
    dst = edge_index[1]
    pad = EP - E
    srcm = jnp.concatenate([src, jnp.zeros((pad,), jnp.int32)]).reshape(RP, G)
    dstm = jnp.concatenate([dst, jnp.full((pad,), N, jnp.int32)]).reshape(RP, G)

    p = _sc_agg(x, srcm, dstm)
    h1 = _mlp(x, p[:N], p[N_PAD:N_PAD + N], W1, b1, W2, b2, relu_out=True)
    p2 = _sc_agg(h1, srcm, dstm)
    return _mlp(h1, p2[:N], p2[N_PAD:N_PAD + N], W3, b3, W4, b4, relu_out=False)


# speedup vs baseline: 1.1180x; 1.1180x over previous
import functools

import jax
import jax.numpy as jnp
from jax import lax
from jax.experimental import pallas as pl
from jax.experimental.pallas import tpu as pltpu
from jax.experimental.pallas import tpu_sc as plsc

N = 10000
E = 320000
D = 128

NC = 2
NS = 16
NW = NC * NS

G = 128
RW = 80
RP = RW * NW
XR = 8
EP = (RP + XR) * G

N_PAD = 10240
ZR = 16
ROWS_PER_SUB = N_PAD // NS


def _sc_agg_body(x_hbm, srcm_hbm, dstm_hbm, out_hbm,
                 s0, d0, s1, d1, m0, m1, zbuf_v, acc, g0, g1):
    c = lax.axis_index("c")
    s = lax.axis_index("s")
    wid = s * NC + c

    zero = jnp.zeros((16,), jnp.float32)
    for i in range(ZR):
        for j in range(D // 16):
            zbuf_v[i, pl.ds(j * 16, 16)] = zero

    def zbody(i, carry):
        pltpu.sync_copy(zbuf_v, acc.at[pl.ds(s * ROWS_PER_SUB + i * ZR, ZR)])
        return carry
    lax.fori_loop(0, ROWS_PER_SUB // ZR, zbody, 0)

    plsc.subcore_barrier()

    base = wid * RW
    pltpu.sync_copy(srcm_hbm.at[base], s0)
    pltpu.sync_copy(dstm_hbm.at[base], d0)
    pltpu.async_copy(x_hbm.at[s0], m0, g0)

    def ebody(t, carry):
        row = base + 2 * t
        pltpu.sync_copy(srcm_hbm.at[row + 1], s1)
        pltpu.sync_copy(dstm_hbm.at[row + 1], d1)
        pltpu.make_async_copy(x_hbm.at[s0], m0, g0).wait()
        pltpu.async_copy(x_hbm.at[s1], m1, g1)
        pltpu.sync_copy(m0, acc.at[d0], add=True)
        pltpu.sync_copy(srcm_hbm.at[row + 2], s0)
        pltpu.sync_copy(dstm_hbm.at[row + 2], d0)
        pltpu.make_async_copy(x_hbm.at[s1], m1, g1).wait()
        pltpu.async_copy(x_hbm.at[s0], m0, g0)
        pltpu.sync_copy(m1, acc.at[d1], add=True)
        return carry
    lax.fori_loop(0, RW // 2, ebody, 0)

    pltpu.make_async_copy(x_hbm.at[s0], m0, g0).wait()

    plsc.subcore_barrier()

    base = c * N_PAD + s * ROWS_PER_SUB
    pltpu.sync_copy(acc.at[pl.ds(s * ROWS_PER_SUB, ROWS_PER_SUB)],
                    out_hbm.at[pl.ds(base, ROWS_PER_SUB)])


def _sc_agg(x, srcm, dstm):
    mesh = plsc.VectorSubcoreMesh(core_axis_name="c", subcore_axis_name="s",
                                  num_cores=NC, num_subcores=NS)
    return pl.kernel(
        _sc_agg_body,
        out_type=jax.ShapeDtypeStruct((NC * N_PAD, D), jnp.float32),
        mesh=mesh,
        scratch_types=[
            pltpu.VMEM((G,), jnp.int32),
            pltpu.VMEM((G,), jnp.int32),
            pltpu.VMEM((G,), jnp.int32),
            pltpu.VMEM((G,), jnp.int32),
            pltpu.VMEM((G, D), jnp.float32),
            pltpu.VMEM((G, D), jnp.float32),
            pltpu.VMEM((ZR, D), jnp.float32),
            pltpu.VMEM_SHARED((N_PAD, D), jnp.float32),
            pltpu.SemaphoreType.DMA,
            pltpu.SemaphoreType.DMA,
        ],
    )(x, srcm, dstm)


def _mlp_body(x_ref, p0_ref, p1_ref, w1_ref, b1_ref, w2_ref, b2_ref, o_ref,
              *, relu_out):
    h = x_ref[...] + p0_ref[...] + p1_ref[...]
    h = jnp.dot(h, w1_ref[...], preferred_element_type=jnp.float32) + b1_ref[...]
    h = jnp.maximum(h, 0.0)
    h = jnp.dot(h, w2_ref[...], preferred_element_type=jnp.float32) + b2_ref[...]
    if relu_out:
        h = jnp.maximum(h, 0.0)
    o_ref[...] = h


def _mlp(x, p0, p1, W_a, b_a, W_b, b_b, relu_out):
    BR = 2000
    row_spec = pl.BlockSpec((BR, D), lambda i: (i, 0))
    full = pl.BlockSpec((D, D), lambda i: (0, 0))
    vec = pl.BlockSpec((1, D), lambda i: (0, 0))
    return pl.pallas_call(
        functools.partial(_mlp_body, relu_out=relu_out),
        grid=(N // BR,),
        in_specs=[row_spec, row_spec, row_spec, full, vec, full, vec],
        out_specs=row_spec,
        out_shape=jax.ShapeDtypeStruct((N, D), jnp.float32),
    )(x, p0, p1, W_a, b_a.reshape(1, D), W_b, b_b.reshape(1, D))


def kernel(x, edge_index, W1, b1, W2, b2, W3, b3, W4, b4):
    src = edge_index[0]
    dst = edge_index[1]
    pad = EP - E
    srcm = jnp.concatenate([src, jnp.zeros((pad,), jnp.int32)]).reshape(-1, G)
    dstm = jnp.concatenate([dst, jnp.full((pad,), N, jnp.int32)]).reshape(-1, G)

    p = _sc_agg(x, srcm, dstm)
    h1 = _mlp(x, p[:N], p[N_PAD:N_PAD + N], W1, b1, W2, b2, relu_out=True)
    p2 = _sc_agg(h1, srcm, dstm)
    return _mlp(h1, p2[:N], p2[N_PAD:N_PAD + N], W3, b3, W4, b4, relu_out=False)


# speedup vs baseline: 1.4343x; 1.2829x over previous
import functools

import jax
import jax.numpy as jnp
from jax import lax
from jax.experimental import pallas as pl
from jax.experimental.pallas import tpu as pltpu
from jax.experimental.pallas import tpu_sc as plsc

N = 10000
E = 320000
D = 128

NC = 2
NS = 16
NW = NC * NS

G = 128
RW = -(-E // (G * NW))
RP = RW * NW
EP = RP * G

N_PAD = 10240
ZR = 16
ROWS_PER_SUB = N_PAD // NS


def _sc_agg_body(x_hbm, srcm_hbm, dstm_hbm, out_hbm,
                 src_v, dst_v, msgs_v, zbuf_v, acc, sem):
    c = lax.axis_index("c")
    s = lax.axis_index("s")
    wid = s * NC + c

    zero = jnp.zeros((16,), jnp.float32)
    for i in range(ZR):
        for j in range(D // 16):
            zbuf_v[i, pl.ds(j * 16, 16)] = zero

    def zbody(i, carry):
        pltpu.sync_copy(zbuf_v, acc.at[pl.ds(s * ROWS_PER_SUB + i * ZR, ZR)])
        return carry
    lax.fori_loop(0, ROWS_PER_SUB // ZR, zbody, 0)

    plsc.subcore_barrier()

    def ebody(j, carry):
        row = wid * RW + j
        pltpu.sync_copy(srcm_hbm.at[row], src_v)
        pltpu.sync_copy(dstm_hbm.at[row], dst_v)
        pltpu.async_copy(x_hbm.at[src_v], msgs_v, sem).wait()
        pltpu.sync_copy(msgs_v, acc.at[dst_v], add=True)
        return carry
    lax.fori_loop(0, RW, ebody, 0)

    plsc.subcore_barrier()

    base = c * N_PAD + s * ROWS_PER_SUB
    pltpu.sync_copy(acc.at[pl.ds(s * ROWS_PER_SUB, ROWS_PER_SUB)],
                    out_hbm.at[pl.ds(base, ROWS_PER_SUB)])


def _sc_agg(x, srcm, dstm):
    mesh = plsc.VectorSubcoreMesh(core_axis_name="c", subcore_axis_name="s",
                                  num_cores=NC, num_subcores=NS)
    return pl.kernel(
        _sc_agg_body,
        out_type=jax.ShapeDtypeStruct((NC * N_PAD, D), jnp.float32),
        mesh=mesh,
        scratch_types=[
            pltpu.VMEM((G,), jnp.int32),
            pltpu.VMEM((G,), jnp.int32),
            pltpu.VMEM((G, D), jnp.float32),
            pltpu.VMEM((ZR, D), jnp.float32),
            pltpu.VMEM_SHARED((N_PAD, D), jnp.float32),
            pltpu.SemaphoreType.DMA,
        ],
    )(x, srcm, dstm)


def _mlp_body(x_ref, p0_ref, p1_ref, w1_ref, b1_ref, w2_ref, b2_ref, o_ref,
              *, relu_out):
    h = x_ref[...] + p0_ref[...] + p1_ref[...]
    h = jnp.dot(h, w1_ref[...], preferred_element_type=jnp.float32) + b1_ref[...]
    h = jnp.maximum(h, 0.0)
    h = jnp.dot(h, w2_ref[...], preferred_element_type=jnp.float32) + b2_ref[...]
    if relu_out:
        h = jnp.maximum(h, 0.0)
    o_ref[...] = h


def _mlp(x, p0, p1, W_a, b_a, W_b, b_b, relu_out):
    BR = 2000
    row_spec = pl.BlockSpec((BR, D), lambda i: (i, 0))
    full = pl.BlockSpec((D, D), lambda i: (0, 0))
    vec = pl.BlockSpec((1, D), lambda i: (0, 0))
    return pl.pallas_call(
        functools.partial(_mlp_body, relu_out=relu_out),
        grid=(N // BR,),
        in_specs=[row_spec, row_spec, row_spec, full, vec, full, vec],
        out_specs=row_spec,
        out_shape=jax.ShapeDtypeStruct((N, D), jnp.float32),
    )(x, p0, p1, W_a, b_a.reshape(1, D), W_b, b_b.reshape(1, D))


def kernel(x, edge_index, W1, b1, W2, b2, W3, b3, W4, b4):
    src = edge_index[0]
    dst = edge_index[1]
    pad = EP - E
    srcm = jnp.concatenate([src, jnp.zeros((pad,), jnp.int32)]).reshape(RP, G)
    dstm = jnp.concatenate([dst, jnp.full((pad,), N, jnp.int32)]).reshape(RP, G)

    p = _sc_agg(x, srcm, dstm)
    h1 = _mlp(x, p[:N], p[N_PAD:N_PAD + N], W1, b1, W2, b2, relu_out=True)
    p2 = _sc_agg(h1, srcm, dstm)
    return _mlp(h1, p2[:N], p2[N_PAD:N_PAD + N], W3, b3, W4, b4, relu_out=False)
